# baseline (speedup 1.0000x reference)
import functools
import math

import jax
import jax.numpy as jnp
from jax import lax
from jax.experimental import pallas as pl
from jax.experimental.pallas import tpu as pltpu
from jax.experimental.pallas import tpu_sc as plsc

D_MODEL = 64
SCALE = math.sqrt(D_MODEL)
NUM_WORKERS = 32
X_ROWS = 4096
X_COLS = 200
B_SLAB = X_ROWS // NUM_WORKERS
NBUF = 4
LANES = 16


def _make_kernel():
    mesh = plsc.VectorSubcoreMesh(core_axis_name="c", subcore_axis_name="s")

    @functools.partial(
        pl.kernel,
        out_type=jax.ShapeDtypeStruct((X_COLS, D_MODEL, X_ROWS), jnp.float32),
        mesh=mesh,
        compiler_params=pltpu.CompilerParams(
            use_tc_tiling_on_sc=False, needs_layout_passes=False
        ),
        scratch_types=(
            [pltpu.VMEM((X_COLS, B_SLAB), jnp.int32)]
            + [pltpu.VMEM((B_SLAB, D_MODEL), jnp.float32)] * NBUF
            + [pltpu.VMEM((D_MODEL, B_SLAB), jnp.float32)] * NBUF
            + [pltpu.SemaphoreType.DMA] * (2 * NBUF)
        ),
    )
    def gather_scale(xt_hbm, table_hbm, out_hbm, xt_v, *bufs_and_sems):
        rows = list(bufs_and_sems[:NBUF])
        obuf = list(bufs_and_sems[NBUF:2 * NBUF])
        gsem = list(bufs_and_sems[2 * NBUF:3 * NBUF])
        osem = list(bufs_and_sems[3 * NBUF:])
        wid = lax.axis_index("s") * 2 + lax.axis_index("c")
        col0 = wid * B_SLAB

        pltpu.sync_copy(xt_hbm.at[:, pl.ds(col0, B_SLAB)], xt_v)

        def gather_desc(t, b):
            src = table_hbm.at[xt_v.at[t]]
            return pltpu.make_async_copy(src, rows[b], gsem[b])

        def out_desc(t, b):
            dst = out_hbm.at[t, :, pl.ds(col0, B_SLAB)]
            return pltpu.make_async_copy(obuf[b], dst, osem[b])

        for t0 in range(NBUF - 1):
            gather_desc(t0, t0).start()

        def quad_body(q, carry):
            for b in range(NBUF):
                t = q * NBUF + b
                gather_desc(t, b).wait()

                @pl.when(t >= NBUF)
                def _wait_prev_out():
                    out_desc(t - NBUF, b).wait()

                lane = lax.iota(jnp.int32, LANES)

                def transpose_body(j, carry2):
                    rids = lane + j * LANES
                    for k in range(D_MODEL // LANES):
                        for s in range(LANES):
                            perm = lax.bitwise_and(lane + s, LANES - 1)
                            dids = perm + k * LANES
                            vals = plsc.load_gather(rows[b], [rids, dids])
                            plsc.store_scatter(obuf[b], [dids, rids],
                                               vals * SCALE)
                    return carry2

                lax.fori_loop(0, B_SLAB // LANES, transpose_body, 0)
                out_desc(t, b).start()

                @pl.when(t + NBUF - 1 < X_COLS)
                def _start_next_gather():
                    gather_desc(t + NBUF - 1, (b + NBUF - 1) % NBUF).start()
            return carry

        lax.fori_loop(0, X_COLS // NBUF, quad_body, 0)
        for b in range(NBUF):
            out_desc(X_COLS - NBUF + b, b).wait()

    return gather_scale


_gather_scale = _make_kernel()


def kernel(x, embed):
    out = _gather_scale(x.T, embed)
    return out.transpose((2, 0, 1))


# speedup vs baseline: 1.0059x; 1.0059x over previous
import functools
import math

import jax
import jax.numpy as jnp
from jax import lax
from jax.experimental import pallas as pl
from jax.experimental.pallas import tpu as pltpu
from jax.experimental.pallas import tpu_sc as plsc

D_MODEL = 64
SCALE = math.sqrt(D_MODEL)
NUM_WORKERS = 32
X_ROWS = 4096
X_COLS = 200
B_SLAB = X_ROWS // NUM_WORKERS
NBUF = 4
LANES = 16


def _make_kernel():
    mesh = plsc.VectorSubcoreMesh(core_axis_name="c", subcore_axis_name="s")

    @functools.partial(
        pl.kernel,
        out_type=jax.ShapeDtypeStruct((X_COLS, D_MODEL, X_ROWS), jnp.float32),
        mesh=mesh,
        compiler_params=pltpu.CompilerParams(
            use_tc_tiling_on_sc=False,
            needs_layout_passes=False,
            disable_bounds_checks=True,
        ),
        scratch_types=(
            [pltpu.VMEM((X_COLS, B_SLAB), jnp.int32)]
            + [pltpu.VMEM((B_SLAB, D_MODEL), jnp.float32)] * NBUF
            + [pltpu.VMEM((D_MODEL, B_SLAB), jnp.float32)] * NBUF
            + [pltpu.SemaphoreType.DMA] * (2 * NBUF)
        ),
    )
    def gather_scale(xt_hbm, table_hbm, out_hbm, xt_v, *bufs_and_sems):
        rows = list(bufs_and_sems[:NBUF])
        obuf = list(bufs_and_sems[NBUF:2 * NBUF])
        gsem = list(bufs_and_sems[2 * NBUF:3 * NBUF])
        osem = list(bufs_and_sems[3 * NBUF:])
        wid = lax.axis_index("s") * 2 + lax.axis_index("c")
        col0 = wid * B_SLAB

        pltpu.sync_copy(xt_hbm.at[:, pl.ds(col0, B_SLAB)], xt_v)

        def gather_desc(t, b):
            src = table_hbm.at[xt_v.at[t]]
            return pltpu.make_async_copy(src, rows[b], gsem[b])

        def out_desc(t, b):
            dst = out_hbm.at[t, :, pl.ds(col0, B_SLAB)]
            return pltpu.make_async_copy(obuf[b], dst, osem[b])

        for t0 in range(NBUF - 1):
            gather_desc(t0, t0).start()

        def quad_body(q, carry):
            for b in range(NBUF):
                t = q * NBUF + b
                gather_desc(t, b).wait()

                @pl.when(t >= NBUF)
                def _wait_prev_out():
                    out_desc(t - NBUF, b).wait()

                lane = lax.iota(jnp.int32, LANES)

                @plsc.parallel_loop(0, B_SLAB // LANES, unroll=1)
                def transpose_body(j):
                    rids = lane + j * LANES
                    for k in range(D_MODEL // LANES):
                        for s in range(LANES):
                            perm = lax.bitwise_and(lane + s, LANES - 1)
                            dids = perm + k * LANES
                            vals = plsc.load_gather(rows[b], [rids, dids])
                            plsc.store_scatter(obuf[b], [dids, rids],
                                               vals * SCALE)
                out_desc(t, b).start()

                @pl.when(t + NBUF - 1 < X_COLS)
                def _start_next_gather():
                    gather_desc(t + NBUF - 1, (b + NBUF - 1) % NBUF).start()
            return carry

        lax.fori_loop(0, X_COLS // NBUF, quad_body, 0)
        for b in range(NBUF):
            out_desc(X_COLS - NBUF + b, b).wait()

    return gather_scale


_gather_scale = _make_kernel()


def kernel(x, embed):
    out = _gather_scale(x.T, embed)
    return out.transpose((2, 0, 1))


# speedup vs baseline: 1.0630x; 1.0568x over previous
import functools
import math

import jax
import jax.numpy as jnp
from jax import lax
from jax.experimental import pallas as pl
from jax.experimental.pallas import tpu as pltpu
from jax.experimental.pallas import tpu_sc as plsc

D_MODEL = 64
SCALE = math.sqrt(D_MODEL)
NUM_WORKERS = 32
X_ROWS = 4096
X_COLS = 200
T_PAIRS = X_COLS // 2
B_SLAB = X_ROWS // NUM_WORKERS
NRING = 2
LANES = 16

BB = 512


def _make_sc_gather():
    mesh = plsc.VectorSubcoreMesh(core_axis_name="c", subcore_axis_name="s")

    @functools.partial(
        pl.kernel,
        out_type=jax.ShapeDtypeStruct((T_PAIRS, X_ROWS, 2 * D_MODEL),
                                      jnp.float32),
        mesh=mesh,
        compiler_params=pltpu.CompilerParams(
            use_tc_tiling_on_sc=False,
            needs_layout_passes=False,
        ),
        scratch_types=(
            [pltpu.VMEM((X_COLS, B_SLAB), jnp.int32)]
            + [pltpu.VMEM((B_SLAB, D_MODEL), jnp.float32)] * (2 * NRING)
            + [pltpu.VMEM((B_SLAB, 2 * D_MODEL), jnp.float32)] * NRING
            + [pltpu.SemaphoreType.DMA] * (3 * NRING)
        ),
    )
    def gather_scale(xt_hbm, table_hbm, mid_hbm, xt_v, *bufs_and_sems):
        rows = list(bufs_and_sems[:2 * NRING])
        obuf = list(bufs_and_sems[2 * NRING:3 * NRING])
        gsem = list(bufs_and_sems[3 * NRING:5 * NRING])
        osem = list(bufs_and_sems[5 * NRING:])
        wid = lax.axis_index("s") * 2 + lax.axis_index("c")
        col0 = wid * B_SLAB

        pltpu.sync_copy(xt_hbm.at[:, pl.ds(col0, B_SLAB)], xt_v)

        def gather_desc(p, e, r):
            src = table_hbm.at[xt_v.at[p * 2 + e]]
            return pltpu.make_async_copy(src, rows[r], gsem[r])

        def out_desc(p, r):
            dst = mid_hbm.at[p, pl.ds(col0, B_SLAB), :]
            return pltpu.make_async_copy(obuf[r], dst, osem[r])

        for e in range(2):
            gather_desc(0, e, e).start()
            gather_desc(1, e, 2 + e).start()

        def pair_body(h, carry):
            for r in range(NRING):
                p = h * NRING + r
                gather_desc(p, 0, 2 * r).wait()
                gather_desc(p, 1, 2 * r + 1).wait()

                @pl.when(p >= NRING)
                def _wait_prev_out():
                    out_desc(p - NRING, r).wait()

                @plsc.parallel_loop(0, B_SLAB, unroll=2)
                def merge_body(q):
                    for e in range(2):
                        for j in range(D_MODEL // LANES):
                            src = pl.ds(j * LANES, LANES)
                            dst = pl.ds(e * D_MODEL + j * LANES, LANES)
                            obuf[r][q, dst] = rows[2 * r + e][q, src] * SCALE

                out_desc(p, r).start()

                @pl.when(p + NRING < T_PAIRS)
                def _start_next_gathers():
                    gather_desc(p + NRING, 0, 2 * r).start()
                    gather_desc(p + NRING, 1, 2 * r + 1).start()
            return carry

        lax.fori_loop(0, T_PAIRS // NRING, pair_body, 0)
        for r in range(NRING):
            out_desc(T_PAIRS - NRING + r, r).wait()

    return gather_scale


def _tc_body(m_ref, o_ref):
    m = m_ref[0]
    for e in range(2):
        o_ref[e] = jnp.transpose(m[:, e * D_MODEL:(e + 1) * D_MODEL], (1, 0))


def _make_tc_transpose():
    return pl.pallas_call(
        _tc_body,
        grid=(T_PAIRS, X_ROWS // BB),
        in_specs=[pl.BlockSpec((1, BB, 2 * D_MODEL), lambda p, c: (p, c, 0))],
        out_specs=pl.BlockSpec((2, D_MODEL, BB), lambda p, c: (p, 0, c)),
        out_shape=jax.ShapeDtypeStruct((X_COLS, D_MODEL, X_ROWS), jnp.float32),
        compiler_params=pltpu.CompilerParams(
            dimension_semantics=("arbitrary", "arbitrary"),
        ),
    )


_sc_gather = _make_sc_gather()
_tc_transpose = _make_tc_transpose()


def kernel(x, embed):
    mid = _sc_gather(x.T, embed)
    out = _tc_transpose(mid)
    return out.transpose((2, 0, 1))
